# baseline (speedup 1.0000x reference)
import functools

import jax
import jax.numpy as jnp
from jax import lax
from jax.experimental import pallas as pl
from jax.experimental.pallas import tpu as pltpu
from jax.experimental.pallas import tpu_sc as plsc

VOCAB = 100000
EMBED = 64
N_FEAT = 36
HIDDEN = 1024
N_CLASSES = 3
BATCH = 4096
FAN_IN = N_FEAT * EMBED

NC = 2
NS = 16
NW = NC * NS
TOTAL = BATCH * N_FEAT

NSLICE = 4
TOT_S = TOTAL // NSLICE
BS = BATCH // NSLICE
PW_S = TOT_S // NW
CHUNK = 128
NCH = PW_S // CHUNK
NRING = 9
LAG = 3
NGRP = NCH // NRING


def _sc_gather(embeddings, idx_flat, s):
    mesh = plsc.VectorSubcoreMesh(core_axis_name="c", subcore_axis_name="s")

    @functools.partial(
        pl.kernel,
        out_type=jax.ShapeDtypeStruct((TOT_S, EMBED), jnp.bfloat16),
        mesh=mesh,
        scratch_types=(
            [pltpu.VMEM((PW_S,), jnp.int32),
             pltpu.VMEM((NRING, CHUNK, EMBED), jnp.bfloat16)]
            + [pltpu.SemaphoreType.DMA] * (2 * NRING)
        ),
        compiler_params=pltpu.CompilerParams(use_tc_tiling_on_sc=False),
    )
    def gather_kernel(table_hbm, idx_hbm, out_hbm, idx_v, rows_v, *sems):
        gsems = sems[:NRING]
        wsems = sems[NRING:]
        wid = lax.axis_index("c") * NS + lax.axis_index("s")
        base = wid * PW_S
        pltpu.sync_copy(idx_hbm.at[pl.ds(s * TOT_S + base, PW_S)], idx_v)

        def fire_gather(j, b):
            pltpu.async_copy(
                table_hbm.at[idx_v.at[pl.ds(j * CHUNK, CHUNK)]],
                rows_v.at[b], gsems[b])

        def drain_gather(b):
            pltpu.make_async_copy(
                table_hbm.at[idx_v.at[pl.ds(0, CHUNK)]],
                rows_v.at[b], gsems[b]).wait()

        def fire_write(j, b):
            pltpu.async_copy(
                rows_v.at[b],
                out_hbm.at[pl.ds(base + j * CHUNK, CHUNK)], wsems[b])

        def drain_write(b):
            pltpu.make_async_copy(
                rows_v.at[b],
                out_hbm.at[pl.ds(base, CHUNK)], wsems[b]).wait()

        for b in range(NRING - LAG):
            fire_gather(b, b)

        def body(g, carry):
            for b in range(NRING):
                j = g * NRING + b
                drain_gather(b)
                fire_write(j, b)
                bb = (b - LAG) % NRING
                jj = j + NRING - LAG

                @pl.when(j >= LAG)
                def _():
                    drain_write(bb)

                @pl.when(jj < NCH)
                def _():
                    fire_gather(jj, bb)

            return carry

        lax.fori_loop(0, NGRP, body, 0)
        for i in range(LAG):
            drain_write((NCH - LAG + i) % NRING)

    return gather_kernel(embeddings, idx_flat)


BM = 512


def _mlp_body(x_ref, w1_ref, b1_ref, w2_ref, b2_ref, o_ref):
    x = x_ref[...]
    h = lax.dot_general(x, w1_ref[...], (((1,), (1,)), ((), ())),
                        preferred_element_type=jnp.float32)
    h = jnp.maximum(h + b1_ref[...], 0.0).astype(jnp.bfloat16)
    o = lax.dot_general(h, w2_ref[...], (((1,), (1,)), ((), ())),
                        preferred_element_type=jnp.float32)
    o_ref[...] = o + b2_ref[...]


def _tc_mlp(x, W1b, b1, W2b, b2):
    grid = (x.shape[0] // BM,)
    return pl.pallas_call(
        _mlp_body,
        grid=grid,
        in_specs=[
            pl.BlockSpec((BM, FAN_IN), lambda i: (i, 0)),
            pl.BlockSpec((HIDDEN, FAN_IN), lambda i: (0, 0)),
            pl.BlockSpec((1, HIDDEN), lambda i: (0, 0)),
            pl.BlockSpec((N_CLASSES, HIDDEN), lambda i: (0, 0)),
            pl.BlockSpec((1, N_CLASSES), lambda i: (0, 0)),
        ],
        out_specs=pl.BlockSpec((BM, N_CLASSES), lambda i: (i, 0)),
        out_shape=jax.ShapeDtypeStruct((x.shape[0], N_CLASSES), jnp.float32),
        compiler_params=pltpu.CompilerParams(
            dimension_semantics=("arbitrary",),
        ),
    )(x, W1b, b1, W2b, b2)


def kernel(t, embeddings, W1, b1, W2, b2):
    idx_flat = t.astype(jnp.int32).reshape(TOTAL)
    table_bf16 = embeddings.astype(jnp.bfloat16)
    W1b = W1.astype(jnp.bfloat16)
    W2b = W2.astype(jnp.bfloat16)
    b1r = b1.reshape(1, HIDDEN)
    b2r = b2.reshape(1, N_CLASSES)
    outs = []
    for s in range(NSLICE):
        xs = _sc_gather(table_bf16, idx_flat, s).reshape(BS, FAN_IN)
        outs.append(_tc_mlp(xs, W1b, b1r, W2b, b2r))
    return jnp.concatenate(outs, axis=0)


# speedup vs baseline: 1.3432x; 1.3432x over previous
import functools

import jax
import jax.numpy as jnp
from jax import lax
from jax.experimental import pallas as pl
from jax.experimental.pallas import tpu as pltpu
from jax.experimental.pallas import tpu_sc as plsc

VOCAB = 100000
EMBED = 64
N_FEAT = 36
HIDDEN = 1024
N_CLASSES = 3
BATCH = 4096
FAN_IN = N_FEAT * EMBED

NC = 2
NS = 16
NW = NC * NS
TOTAL = BATCH * N_FEAT

NSLICE = 4
TOT_S = TOTAL // NSLICE
BS = BATCH // NSLICE
PW_S = TOT_S // NW
CHUNK = 128
NCH = PW_S // CHUNK
NRING = 9
LAG = 3
NGRP = NCH // NRING


def _sc_gather(embeddings, idx_flat, s):
    mesh = plsc.VectorSubcoreMesh(core_axis_name="c", subcore_axis_name="s")

    @functools.partial(
        pl.kernel,
        out_type=jax.ShapeDtypeStruct((TOT_S, EMBED), jnp.float32),
        mesh=mesh,
        scratch_types=(
            [pltpu.VMEM((PW_S,), jnp.int32),
             pltpu.VMEM((NRING, CHUNK, EMBED), jnp.float32)]
            + [pltpu.SemaphoreType.DMA] * (2 * NRING)
        ),
        compiler_params=pltpu.CompilerParams(use_tc_tiling_on_sc=False),
    )
    def gather_kernel(table_hbm, idx_hbm, out_hbm, idx_v, rows_v, *sems):
        gsems = sems[:NRING]
        wsems = sems[NRING:]
        wid = lax.axis_index("c") * NS + lax.axis_index("s")
        base = wid * PW_S
        pltpu.sync_copy(idx_hbm.at[pl.ds(s * TOT_S + base, PW_S)], idx_v)

        def fire_gather(j, b):
            pltpu.async_copy(
                table_hbm.at[idx_v.at[pl.ds(j * CHUNK, CHUNK)]],
                rows_v.at[b], gsems[b])

        def drain_gather(b):
            pltpu.make_async_copy(
                table_hbm.at[idx_v.at[pl.ds(0, CHUNK)]],
                rows_v.at[b], gsems[b]).wait()

        def fire_write(j, b):
            pltpu.async_copy(
                rows_v.at[b],
                out_hbm.at[pl.ds(base + j * CHUNK, CHUNK)], wsems[b])

        def drain_write(b):
            pltpu.make_async_copy(
                rows_v.at[b],
                out_hbm.at[pl.ds(base, CHUNK)], wsems[b]).wait()

        for b in range(NRING - LAG):
            fire_gather(b, b)

        def body(g, carry):
            for b in range(NRING):
                j = g * NRING + b
                drain_gather(b)
                fire_write(j, b)
                bb = (b - LAG) % NRING
                jj = j + NRING - LAG

                @pl.when(j >= LAG)
                def _():
                    drain_write(bb)

                @pl.when(jj < NCH)
                def _():
                    fire_gather(jj, bb)

            return carry

        lax.fori_loop(0, NGRP, body, 0)
        for i in range(LAG):
            drain_write((NCH - LAG + i) % NRING)

    return gather_kernel(embeddings, idx_flat)


BM = 512


def _mlp_body(x_ref, w1_ref, b1_ref, w2_ref, b2_ref, o_ref):
    x = x_ref[...].astype(jnp.bfloat16)
    h = lax.dot_general(x, w1_ref[...], (((1,), (1,)), ((), ())),
                        preferred_element_type=jnp.float32)
    h = jnp.maximum(h + b1_ref[...], 0.0).astype(jnp.bfloat16)
    o = lax.dot_general(h, w2_ref[...], (((1,), (1,)), ((), ())),
                        preferred_element_type=jnp.float32)
    o_ref[...] = o + b2_ref[...]


def _tc_mlp(x, W1b, b1, W2b, b2):
    grid = (x.shape[0] // BM,)
    return pl.pallas_call(
        _mlp_body,
        grid=grid,
        in_specs=[
            pl.BlockSpec((BM, FAN_IN), lambda i: (i, 0)),
            pl.BlockSpec((HIDDEN, FAN_IN), lambda i: (0, 0)),
            pl.BlockSpec((1, HIDDEN), lambda i: (0, 0)),
            pl.BlockSpec((N_CLASSES, HIDDEN), lambda i: (0, 0)),
            pl.BlockSpec((1, N_CLASSES), lambda i: (0, 0)),
        ],
        out_specs=pl.BlockSpec((BM, N_CLASSES), lambda i: (i, 0)),
        out_shape=jax.ShapeDtypeStruct((x.shape[0], N_CLASSES), jnp.float32),
        compiler_params=pltpu.CompilerParams(
            dimension_semantics=("arbitrary",),
        ),
    )(x, W1b, b1, W2b, b2)


def kernel(t, embeddings, W1, b1, W2, b2):
    idx_flat = t.astype(jnp.int32).reshape(TOTAL)
    W1b = W1.astype(jnp.bfloat16)
    W2b = W2.astype(jnp.bfloat16)
    b1r = b1.reshape(1, HIDDEN)
    b2r = b2.reshape(1, N_CLASSES)
    outs = []
    for s in range(NSLICE):
        xs = _sc_gather(embeddings, idx_flat, s).reshape(BS, FAN_IN)
        outs.append(_tc_mlp(xs, W1b, b1r, W2b, b2r))
    return jnp.concatenate(outs, axis=0)


# speedup vs baseline: 1.4560x; 1.0840x over previous
import functools

import jax
import jax.numpy as jnp
from jax import lax
from jax.experimental import pallas as pl
from jax.experimental.pallas import tpu as pltpu
from jax.experimental.pallas import tpu_sc as plsc

VOCAB = 100000
EMBED = 64
N_FEAT = 36
HIDDEN = 1024
N_CLASSES = 3
BATCH = 4096
FAN_IN = N_FEAT * EMBED

NC = 2
NS = 16
NW = NC * NS
TOTAL = BATCH * N_FEAT
PER_W = TOTAL // NW
CHUNK = 256
NCH = PER_W // CHUNK
NRING = 6
LAG = 2
NGRP = NCH // NRING


def _sc_gather(embeddings, idx_flat):
    mesh = plsc.VectorSubcoreMesh(core_axis_name="c", subcore_axis_name="s")

    @functools.partial(
        pl.kernel,
        out_type=jax.ShapeDtypeStruct((TOTAL, EMBED), jnp.float32),
        mesh=mesh,
        scratch_types=(
            [pltpu.VMEM((PER_W,), jnp.int32),
             pltpu.VMEM((NRING, CHUNK, EMBED), jnp.float32)]
            + [pltpu.SemaphoreType.DMA] * (2 * NRING)
        ),
        compiler_params=pltpu.CompilerParams(use_tc_tiling_on_sc=False),
    )
    def gather_kernel(table_hbm, idx_hbm, out_hbm, idx_v, rows_v, *sems):
        gsems = sems[:NRING]
        wsems = sems[NRING:]
        wid = lax.axis_index("c") * NS + lax.axis_index("s")
        base = wid * PER_W
        pltpu.sync_copy(idx_hbm.at[pl.ds(base, PER_W)], idx_v)

        def fire_gather(j, b):
            pltpu.async_copy(
                table_hbm.at[idx_v.at[pl.ds(j * CHUNK, CHUNK)]],
                rows_v.at[b], gsems[b])

        def drain_gather(b):
            pltpu.make_async_copy(
                table_hbm.at[idx_v.at[pl.ds(0, CHUNK)]],
                rows_v.at[b], gsems[b]).wait()

        def fire_write(j, b):
            pltpu.async_copy(
                rows_v.at[b],
                out_hbm.at[pl.ds(base + j * CHUNK, CHUNK)], wsems[b])

        def drain_write(b):
            pltpu.make_async_copy(
                rows_v.at[b],
                out_hbm.at[pl.ds(base, CHUNK)], wsems[b]).wait()

        for b in range(NRING - LAG):
            fire_gather(b, b)

        def body(g, carry):
            for b in range(NRING):
                j = g * NRING + b
                drain_gather(b)
                fire_write(j, b)
                bb = (b - LAG) % NRING
                jj = j + NRING - LAG

                @pl.when(j >= LAG)
                def _():
                    drain_write(bb)

                @pl.when(jj < NCH)
                def _():
                    fire_gather(jj, bb)

            return carry

        lax.fori_loop(0, NGRP, body, 0)
        for i in range(LAG):
            drain_write((NCH - LAG + i) % NRING)

    return gather_kernel(embeddings, idx_flat)


BM = 512


def _mlp_body(x_ref, w1_ref, b1_ref, w2_ref, b2_ref, o_ref):
    x = x_ref[...].astype(jnp.bfloat16)
    h = lax.dot_general(x, w1_ref[...], (((1,), (1,)), ((), ())),
                        preferred_element_type=jnp.float32)
    h = jnp.maximum(h + b1_ref[...], 0.0).astype(jnp.bfloat16)
    o = lax.dot_general(h, w2_ref[...], (((1,), (1,)), ((), ())),
                        preferred_element_type=jnp.float32)
    o_ref[...] = o + b2_ref[...]


def _tc_mlp(x, W1b, b1, W2b, b2):
    grid = (BATCH // BM,)
    return pl.pallas_call(
        _mlp_body,
        grid=grid,
        in_specs=[
            pl.BlockSpec((BM, FAN_IN), lambda i: (i, 0)),
            pl.BlockSpec((HIDDEN, FAN_IN), lambda i: (0, 0)),
            pl.BlockSpec((1, HIDDEN), lambda i: (0, 0)),
            pl.BlockSpec((N_CLASSES, HIDDEN), lambda i: (0, 0)),
            pl.BlockSpec((1, N_CLASSES), lambda i: (0, 0)),
        ],
        out_specs=pl.BlockSpec((BM, N_CLASSES), lambda i: (i, 0)),
        out_shape=jax.ShapeDtypeStruct((BATCH, N_CLASSES), jnp.float32),
        compiler_params=pltpu.CompilerParams(
            dimension_semantics=("arbitrary",),
        ),
    )(x, W1b, b1, W2b, b2)


def kernel(t, embeddings, W1, b1, W2, b2):
    idx_flat = t.astype(jnp.int32).reshape(TOTAL)
    x = _sc_gather(embeddings, idx_flat).reshape(BATCH, FAN_IN)
    W1b = W1.astype(jnp.bfloat16)
    W2b = W2.astype(jnp.bfloat16)
    return _tc_mlp(x, W1b, b1.reshape(1, HIDDEN), W2b, b2.reshape(1, N_CLASSES))
